# Initial kernel scaffold; baseline (speedup 1.0000x reference)
import jax
import jax.numpy as jnp
from jax.experimental import pallas as pl


def kernel(words, bigrams, trigrams, emb1, emb2, emb3, fc1_w, fc1_b, fc2_w, fc2_b):
    raise NotImplementedError("write your pallas kernel here")


# speedup vs baseline: 1.6382x; 1.6382x over previous
import functools

import jax
import jax.numpy as jnp
from jax import lax
from jax.experimental import pallas as pl
from jax.experimental.pallas import tpu as pltpu
from jax.experimental.pallas import tpu_sc as plsc

B = 4096
L = 200
D = 300
DP = 304
OUTW = 912
H1 = 256
NUM_CLASSES = 1000

NC = 2
NS = 16
NW = NC * NS
BPW = B // NW
G = 8
C1, C2 = 104, 96

OFFS = tuple(range(0, DP, 16))
NACC = len(OFFS)

@functools.cache
def _get_pool():
    mesh = plsc.VectorSubcoreMesh(core_axis_name="c", subcore_axis_name="s")

    @functools.partial(
        pl.kernel,
        out_type=jax.ShapeDtypeStruct((B, OUTW), jnp.float32),
        mesh=mesh,
        scratch_types=[
            pltpu.VMEM((C1,), jnp.int32),
            pltpu.VMEM((C2,), jnp.int32),
            pltpu.VMEM((C1, DP), jnp.float32),
            pltpu.VMEM((C2, DP), jnp.float32),
            pltpu.VMEM((G, OUTW), jnp.float32),
            pltpu.SemaphoreType.DMA,
            pltpu.SemaphoreType.DMA,
        ],
        compiler_params=pltpu.CompilerParams(use_tc_tiling_on_sc=False),
    )
    def _pool(words_hbm, bigrams_hbm, trigrams_hbm, e1, e2, e3, out_hbm,
              cidx_a, cidx_b, rows_a, rows_b, out_v, sem_a, sem_b):
        _pool_body(words_hbm, bigrams_hbm, trigrams_hbm, e1, e2, e3, out_hbm,
                   cidx_a, cidx_b, rows_a, rows_b, out_v, sem_a, sem_b)

    return _pool


def _pool_body(words_hbm, bigrams_hbm, trigrams_hbm, e1, e2, e3, out_hbm,
               cidx_a, cidx_b, rows_a, rows_b, out_v, sem_a, sem_b):
    wid = lax.axis_index("s") * NC + lax.axis_index("c")

    def accum(rows_ref, n, accs):
        def body(r, a):
            return tuple(a[i] + rows_ref[r, pl.ds(OFFS[i], 16)]
                         for i in range(NACC))
        return lax.fori_loop(0, n, body, accs)

    def group_body(grp, carry):
        base = wid * BPW + grp * G

        def sample_body(s, carry2):
            el = pl.multiple_of((base + s) * L, 8)
            el_b = pl.multiple_of((base + s) * L + C1, 8)
            for t, (src, tab) in enumerate(
                    ((words_hbm, e1), (bigrams_hbm, e2), (trigrams_hbm, e3))):
                pltpu.sync_copy(src.at[pl.ds(el, C1)], cidx_a)
                pltpu.sync_copy(src.at[pl.ds(el_b, C2)], cidx_b)
                cp_a = pltpu.async_copy(tab.at[cidx_a], rows_a, sem_a)
                cp_b = pltpu.async_copy(tab.at[cidx_b], rows_b, sem_b)
                cp_a.wait()
                cp_b.wait()
                accs = tuple(jnp.zeros((16,), jnp.float32)
                             for _ in range(NACC))
                accs = accum(rows_a, C1, accs)
                accs = accum(rows_b, C2, accs)
                for i in range(NACC):
                    out_v[s, pl.ds(t * D + OFFS[i], 16)] = accs[i]
            return carry2

        lax.fori_loop(0, G, sample_body, 0)
        pltpu.sync_copy(out_v, out_hbm.at[pl.ds(pl.multiple_of(base, 8), G)])
        return carry

    lax.fori_loop(0, BPW // G, group_body, 0)


def _mlp_body(x_ref, w1_ref, b1_ref, w2_ref, b2_ref, o_ref):
    x = x_ref[...] * (1.0 / L)
    h = jnp.dot(x, w1_ref[...], preferred_element_type=jnp.float32)
    h = jnp.maximum(h + b1_ref[...], 0.0)
    o = jnp.dot(h, w2_ref[...], preferred_element_type=jnp.float32)
    o_ref[...] = o + b2_ref[...]


def _mlp(pooled, fc1_w, fc1_b, fc2_w, fc2_b):
    bm = 512
    return pl.pallas_call(
        _mlp_body,
        grid=(B // bm,),
        in_specs=[
            pl.BlockSpec((bm, 3 * D), lambda i: (i, 0)),
            pl.BlockSpec((3 * D, H1), lambda i: (0, 0)),
            pl.BlockSpec((1, H1), lambda i: (0, 0)),
            pl.BlockSpec((H1, NUM_CLASSES), lambda i: (0, 0)),
            pl.BlockSpec((1, NUM_CLASSES), lambda i: (0, 0)),
        ],
        out_specs=pl.BlockSpec((bm, NUM_CLASSES), lambda i: (i, 0)),
        out_shape=jax.ShapeDtypeStruct((B, NUM_CLASSES), jnp.float32),
    )(pooled, fc1_w, fc1_b.reshape(1, H1), fc2_w, fc2_b.reshape(1, NUM_CLASSES))


def kernel(words, bigrams, trigrams, emb1, emb2, emb3,
           fc1_w, fc1_b, fc2_w, fc2_b):
    pad = ((0, 0), (0, DP - D))
    pooled = _get_pool()(words.astype(jnp.int32).reshape(B * L),
                         bigrams.astype(jnp.int32).reshape(B * L),
                         trigrams.astype(jnp.int32).reshape(B * L),
                         jnp.pad(emb1, pad), jnp.pad(emb2, pad),
                         jnp.pad(emb3, pad))
    return _mlp(pooled[:, :3 * D], fc1_w, fc1_b, fc2_w, fc2_b)


# speedup vs baseline: 1.9211x; 1.1727x over previous
import functools

import jax
import jax.numpy as jnp
from jax import lax
from jax.experimental import pallas as pl
from jax.experimental.pallas import tpu as pltpu
from jax.experimental.pallas import tpu_sc as plsc

B = 4096
L = 200
D = 300
DP = 304
OUTW = 912
H1 = 256
NUM_CLASSES = 1000

NC = 2
NS = 16
NW = NC * NS
BPW = B // NW
G = 8
C1, C2 = 104, 96

OFFS = tuple(range(0, DP, 16))
NACC = len(OFFS)


@functools.cache
def _get_pool():
    mesh = plsc.VectorSubcoreMesh(core_axis_name="c", subcore_axis_name="s")

    @functools.partial(
        pl.kernel,
        out_type=jax.ShapeDtypeStruct((B, OUTW), jnp.float32),
        mesh=mesh,
        scratch_types=[
            pltpu.VMEM((C1,), jnp.int32),
            pltpu.VMEM((C2,), jnp.int32),
            pltpu.VMEM((C1, DP), jnp.float32),
            pltpu.VMEM((C2, DP), jnp.float32),
            pltpu.VMEM((G, OUTW), jnp.float32),
            pltpu.SemaphoreType.DMA,
            pltpu.SemaphoreType.DMA,
        ],
        compiler_params=pltpu.CompilerParams(use_tc_tiling_on_sc=False),
    )
    def _pool(words_hbm, bigrams_hbm, trigrams_hbm, e1, e2, e3, out_hbm,
              cidx_a, cidx_b, rows_a, rows_b, out_v, sem_a, sem_b):
        _pool_body(words_hbm, bigrams_hbm, trigrams_hbm, e1, e2, e3, out_hbm,
                   cidx_a, cidx_b, rows_a, rows_b, out_v, sem_a, sem_b)

    return _pool


def _pool_body(words_hbm, bigrams_hbm, trigrams_hbm, e1, e2, e3, out_hbm,
               cidx_a, cidx_b, rows_a, rows_b, out_v, sem_a, sem_b):
    wid = lax.axis_index("s") * NC + lax.axis_index("c")

    def accum(rows_ref, n, accs):
        def body(r, a):
            return tuple(a[i] + rows_ref[r, pl.ds(OFFS[i], 16)]
                         for i in range(NACC))
        return lax.fori_loop(0, n, body, accs)

    def group_body(grp, carry):
        base = wid * BPW + grp * G

        def sample_body(s, carry2):
            el = pl.multiple_of((base + s) * L, 8)
            el_b = pl.multiple_of((base + s) * L + C1, 8)
            for t, (src, tab) in enumerate(
                    ((words_hbm, e1), (bigrams_hbm, e2), (trigrams_hbm, e3))):
                pltpu.sync_copy(src.at[pl.ds(el, C1)], cidx_a)
                pltpu.sync_copy(src.at[pl.ds(el_b, C2)], cidx_b)
                cp_a = pltpu.async_copy(tab.at[cidx_a], rows_a, sem_a)
                cp_b = pltpu.async_copy(tab.at[cidx_b], rows_b, sem_b)
                cp_a.wait()
                cp_b.wait()
                accs = tuple(jnp.zeros((16,), jnp.float32)
                             for _ in range(NACC))
                accs = accum(rows_a, C1, accs)
                accs = accum(rows_b, C2, accs)
                for i in range(NACC):
                    out_v[s, pl.ds(t * D + OFFS[i], 16)] = accs[i]
            return carry2

        lax.fori_loop(0, G, sample_body, 0)
        pltpu.sync_copy(out_v, out_hbm.at[pl.ds(pl.multiple_of(base, 8), G)])
        return carry

    lax.fori_loop(0, BPW // G, group_body, 0)


def _pad_body(x_ref, o_ref):
    o_ref[...] = jnp.concatenate(
        [x_ref[...], jnp.zeros((x_ref.shape[0], DP - D), jnp.float32)],
        axis=1)


def _pad_table(e):
    v = e.shape[0]
    bm = 800
    return pl.pallas_call(
        _pad_body,
        grid=(v // bm,),
        in_specs=[pl.BlockSpec((bm, D), lambda i: (i, 0))],
        out_specs=pl.BlockSpec((bm, DP), lambda i: (i, 0)),
        out_shape=jax.ShapeDtypeStruct((v, DP), jnp.float32),
    )(e)


def _mlp_body(x_ref, w1_ref, b1_ref, w2_ref, b2_ref, o_ref):
    x = x_ref[...] * (1.0 / L)
    h = jnp.dot(x, w1_ref[...], preferred_element_type=jnp.float32)
    h = jnp.maximum(h + b1_ref[...], 0.0)
    o = jnp.dot(h, w2_ref[...], preferred_element_type=jnp.float32)
    o_ref[...] = o + b2_ref[...]


def _mlp(pooled, fc1_w, fc1_b, fc2_w, fc2_b):
    bm = 512
    return pl.pallas_call(
        _mlp_body,
        grid=(B // bm,),
        in_specs=[
            pl.BlockSpec((bm, 3 * D), lambda i: (i, 0)),
            pl.BlockSpec((3 * D, H1), lambda i: (0, 0)),
            pl.BlockSpec((1, H1), lambda i: (0, 0)),
            pl.BlockSpec((H1, NUM_CLASSES), lambda i: (0, 0)),
            pl.BlockSpec((1, NUM_CLASSES), lambda i: (0, 0)),
        ],
        out_specs=pl.BlockSpec((bm, NUM_CLASSES), lambda i: (i, 0)),
        out_shape=jax.ShapeDtypeStruct((B, NUM_CLASSES), jnp.float32),
    )(pooled, fc1_w, fc1_b.reshape(1, H1), fc2_w, fc2_b.reshape(1, NUM_CLASSES))


def kernel(words, bigrams, trigrams, emb1, emb2, emb3,
           fc1_w, fc1_b, fc2_w, fc2_b):
    pooled = _get_pool()(words.astype(jnp.int32).reshape(B * L),
                         bigrams.astype(jnp.int32).reshape(B * L),
                         trigrams.astype(jnp.int32).reshape(B * L),
                         _pad_table(emb1), _pad_table(emb2),
                         _pad_table(emb3))
    return _mlp(pooled[:, :3 * D], fc1_w, fc1_b, fc2_w, fc2_b)


# speedup vs baseline: 2.4677x; 1.2845x over previous
import functools

import jax
import jax.numpy as jnp
from jax import lax
from jax.experimental import pallas as pl
from jax.experimental.pallas import tpu as pltpu
from jax.experimental.pallas import tpu_sc as plsc

B = 4096
L = 200
D = 300
DP = 304
OUTW = 912
H1 = 256
NUM_CLASSES = 1000

NC = 2
NS = 16
NW = NC * NS
BPW = B // NW
G = 8
C1, C2 = 104, 96

OFFS = tuple(range(0, DP, 16))
NACC = len(OFFS)


@functools.cache
def _get_pool():
    mesh = plsc.VectorSubcoreMesh(core_axis_name="c", subcore_axis_name="s")

    @functools.partial(
        pl.kernel,
        out_type=jax.ShapeDtypeStruct((B, OUTW), jnp.float32),
        mesh=mesh,
        scratch_types=[
            pltpu.VMEM((G * L,), jnp.int32),
            pltpu.VMEM((G * L,), jnp.int32),
            pltpu.VMEM((G * L,), jnp.int32),
            pltpu.VMEM((C1, DP), jnp.float32),
            pltpu.VMEM((C1, DP), jnp.float32),
            pltpu.VMEM((G, OUTW), jnp.float32),
            pltpu.SemaphoreType.DMA,
            pltpu.SemaphoreType.DMA,
        ],
        compiler_params=pltpu.CompilerParams(use_tc_tiling_on_sc=False),
    )
    def _pool(words_hbm, bigrams_hbm, trigrams_hbm, e1, e2, e3, out_hbm,
              widx, bidx, tidx, rows0, rows1, out_v, sem0, sem1):
        _pool_body(words_hbm, bigrams_hbm, trigrams_hbm, e1, e2, e3, out_hbm,
                   widx, bidx, tidx, rows0, rows1, out_v, sem0, sem1)

    return _pool


def _pool_body(words_hbm, bigrams_hbm, trigrams_hbm, e1, e2, e3, out_hbm,
               widx, bidx, tidx, rows0, rows1, out_v, sem0, sem1):
    wid = lax.axis_index("s") * NC + lax.axis_index("c")
    rows = (rows0, rows1)
    sems = (sem0, sem1)
    UNROLL = 2

    units = []
    for s in range(G):
        for t in range(3):
            units.append((s, t, s * L, C1))
            units.append((s, t, s * L + C1, C2))

    def fire(u, idx_refs, tabs):
        s, t, off, n = units[u]
        p = u % 2
        return pltpu.async_copy(
            tabs[t].at[idx_refs[t].at[pl.ds(off, n)]],
            rows[p].at[pl.ds(0, n)], sems[p])

    def accum(rows_ref, n, accs):
        def body(r, a):
            new = list(a)
            for rr in range(UNROLL):
                new = [new[i] + rows_ref[r * UNROLL + rr, pl.ds(OFFS[i], 16)]
                       for i in range(NACC)]
            return tuple(new)
        return lax.fori_loop(0, n // UNROLL, body, accs)

    def group_body(grp, carry):
        base = wid * BPW + grp * G
        base_el = pl.multiple_of(base * L, 8)
        pltpu.sync_copy(words_hbm.at[pl.ds(base_el, G * L)], widx)
        pltpu.sync_copy(bigrams_hbm.at[pl.ds(base_el, G * L)], bidx)
        pltpu.sync_copy(trigrams_hbm.at[pl.ds(base_el, G * L)], tidx)
        idx_refs = (widx, bidx, tidx)
        tabs = (e1, e2, e3)

        cps = [None] * len(units)
        cps[0] = fire(0, idx_refs, tabs)
        accs = tuple(jnp.zeros((16,), jnp.float32) for _ in range(NACC))
        for u, (s, t, off, n) in enumerate(units):
            if u + 1 < len(units):
                cps[u + 1] = fire(u + 1, idx_refs, tabs)
            cps[u].wait()
            accs = accum(rows[u % 2], n, accs)
            if n == C2:
                for i in range(NACC):
                    out_v[s, pl.ds(t * D + OFFS[i], 16)] = accs[i]
                accs = tuple(jnp.zeros((16,), jnp.float32)
                             for _ in range(NACC))
        pltpu.sync_copy(out_v, out_hbm.at[pl.ds(pl.multiple_of(base, 8), G)])
        return carry

    lax.fori_loop(0, BPW // G, group_body, 0)


def _pad_body(x_ref, o_ref):
    o_ref[...] = jnp.concatenate(
        [x_ref[...], jnp.zeros((x_ref.shape[0], DP - D), jnp.float32)],
        axis=1)


def _pad_table(e):
    v = e.shape[0]
    bm = 800
    return pl.pallas_call(
        _pad_body,
        grid=(v // bm,),
        in_specs=[pl.BlockSpec((bm, D), lambda i: (i, 0))],
        out_specs=pl.BlockSpec((bm, DP), lambda i: (i, 0)),
        out_shape=jax.ShapeDtypeStruct((v, DP), jnp.float32),
    )(e)


def _mlp_body(x_ref, w1_ref, b1_ref, w2_ref, b2_ref, o_ref):
    x = x_ref[...] * (1.0 / L)
    h = jnp.dot(x, w1_ref[...], preferred_element_type=jnp.float32)
    h = jnp.maximum(h + b1_ref[...], 0.0)
    o = jnp.dot(h, w2_ref[...], preferred_element_type=jnp.float32)
    o_ref[...] = o + b2_ref[...]


def _mlp(pooled, fc1_w, fc1_b, fc2_w, fc2_b):
    bm = 512
    return pl.pallas_call(
        _mlp_body,
        grid=(B // bm,),
        in_specs=[
            pl.BlockSpec((bm, 3 * D), lambda i: (i, 0)),
            pl.BlockSpec((3 * D, H1), lambda i: (0, 0)),
            pl.BlockSpec((1, H1), lambda i: (0, 0)),
            pl.BlockSpec((H1, NUM_CLASSES), lambda i: (0, 0)),
            pl.BlockSpec((1, NUM_CLASSES), lambda i: (0, 0)),
        ],
        out_specs=pl.BlockSpec((bm, NUM_CLASSES), lambda i: (i, 0)),
        out_shape=jax.ShapeDtypeStruct((B, NUM_CLASSES), jnp.float32),
    )(pooled, fc1_w, fc1_b.reshape(1, H1), fc2_w, fc2_b.reshape(1, NUM_CLASSES))


def kernel(words, bigrams, trigrams, emb1, emb2, emb3,
           fc1_w, fc1_b, fc2_w, fc2_b):
    pooled = _get_pool()(words.astype(jnp.int32).reshape(B * L),
                         bigrams.astype(jnp.int32).reshape(B * L),
                         trigrams.astype(jnp.int32).reshape(B * L),
                         _pad_table(emb1), _pad_table(emb2),
                         _pad_table(emb3))
    return _mlp(pooled[:, :3 * D], fc1_w, fc1_b, fc2_w, fc2_b)
